# Initial kernel scaffold; baseline (speedup 1.0000x reference)
import jax
import jax.numpy as jnp
from jax.experimental import pallas as pl


def kernel(preds, targets):
    raise NotImplementedError("write your pallas kernel here")


# speedup vs baseline: 8.6153x; 8.6153x over previous
import functools

import jax
import jax.numpy as jnp
from jax import lax
from jax.experimental import pallas as pl
from jax.experimental.pallas import tpu as pltpu
from jax.experimental.pallas import tpu_sc as plsc

B = 16
C = 19
H = 256
W = 256
TH = 1024
TW = 1024
N_TOTAL = B * C * H * W

NW = 32
ROWS_PER_WORKER = H // 2
CHUNK_ROWS = 8
CHUNKS = ROWS_PER_WORKER // CHUNK_ROWS
CELLS_PER_CHUNK = CHUNK_ROWS * W
PRED_CHUNK = CELLS_PER_CHUNK * C


def _sc_body(preds_hbm, tgt_hbm, out_hbm, tgt_buf, prd_buf, acc_buf, sem):
    cid = lax.axis_index("c")
    sid = lax.axis_index("s")
    wid = cid * 16 + sid
    b = wid // 2
    half = wid % 2

    lane = lax.iota(jnp.int32, 16)

    def chunk_body(ci, acc):
        row0 = half * ROWS_PER_WORKER + ci * CHUNK_ROWS
        copies = []
        for r in range(CHUNK_ROWS):
            copies.append(
                pltpu.make_async_copy(
                    tgt_hbm.at[b, (row0 + r) * 4],
                    tgt_buf.at[pl.ds(r * TW, TW)],
                    sem,
                )
            )
        base = (b * (H * W) + row0 * W) * C
        copies.append(
            pltpu.make_async_copy(
                preds_hbm.at[pl.ds(base, PRED_CHUNK)], prd_buf, sem
            )
        )
        for cp in copies:
            cp.start()
        for cp in copies:
            cp.wait()

        def j_body(j, acc_in):
            cell = j * 16 + lane
            r = lax.shift_right_logical(cell, 8)
            w = lax.bitwise_and(cell, 255)
            cls = plsc.load_gather(tgt_buf, [r * TW + w * 4])
            pidx = cell * C + cls
            x = plsc.load_gather(prd_buf, [pidx])
            term = jnp.minimum(jnp.maximum(-x, -100.0), 100.0)
            return acc_in + term

        return lax.fori_loop(0, CELLS_PER_CHUNK // 16, j_body, acc)

    acc = lax.fori_loop(0, CHUNKS, chunk_body, jnp.zeros((16,), jnp.float32))
    acc_buf[...] = acc
    pltpu.sync_copy(acc_buf, out_hbm.at[wid])


def _sc_sparse_sum(preds_flat, targets):
    mesh = plsc.VectorSubcoreMesh(core_axis_name="c", subcore_axis_name="s")
    f = pl.kernel(
        _sc_body,
        mesh=mesh,
        out_type=jax.ShapeDtypeStruct((NW, 16), jnp.float32),
        scratch_types=[
            pltpu.VMEM((CHUNK_ROWS * TW,), jnp.int32),
            pltpu.VMEM((PRED_CHUNK,), jnp.float32),
            pltpu.VMEM((16,), jnp.float32),
            pltpu.SemaphoreType.DMA,
        ],
        compiler_params=pltpu.CompilerParams(needs_layout_passes=False),
    )
    return f(preds_flat, targets)


def _tc_body(x_ref, o_ref):
    i = pl.program_id(0)
    x = x_ref[...]
    p = jax.nn.sigmoid(x)
    t = jnp.maximum(jnp.log1p(-p), -100.0)
    s = -jnp.sum(t)

    @pl.when(i == 0)
    def _():
        o_ref[0, 0] = s

    @pl.when(i != 0)
    def _():
        o_ref[0, 0] = o_ref[0, 0] + s


def _tc_dense_sum(preds2d):
    return pl.pallas_call(
        _tc_body,
        grid=(C,),
        in_specs=[pl.BlockSpec((B, H * W), lambda i: (i, 0))],
        out_specs=pl.BlockSpec(
            (1, 1), lambda i: (0, 0), memory_space=pltpu.SMEM
        ),
        out_shape=jax.ShapeDtypeStruct((1, 1), jnp.float32),
    )(preds2d)


def kernel(preds, targets):
    targets = targets.astype(jnp.int32)
    preds_flat = preds.reshape(-1)
    sc_parts = _sc_sparse_sum(preds_flat, targets)
    tc_parts = _tc_dense_sum(preds.reshape(B * C, H * W))
    total = jnp.sum(tc_parts) + jnp.sum(sc_parts)
    return total / jnp.float32(N_TOTAL)
